# baseline (speedup 1.0000x reference)
import functools

import jax
import jax.numpy as jnp
from jax import lax
from jax.experimental import pallas as pl
from jax.experimental.pallas import tpu as pltpu
from jax.experimental.pallas import tpu_sc as plsc

B = 4096
D = 1024
K = 4
KP = 8
KM_ITERS = 10


def _feat_body(x_ref, we_ref, be_ref, f_ref):
    f_ref[...] = jnp.maximum(
        jnp.dot(x_ref[...], we_ref[...], preferred_element_type=jnp.float32)
        + be_ref[...],
        0.0,
    )


_FM = 512


def _tc_feat(x, W_ext, b_ext2):
    return pl.pallas_call(
        _feat_body,
        grid=(B // _FM,),
        in_specs=[
            pl.BlockSpec((_FM, D), lambda i: (i, 0)),
            pl.BlockSpec((D, D), lambda i: (0, 0)),
            pl.BlockSpec((1, D), lambda i: (0, 0)),
        ],
        out_specs=pl.BlockSpec((_FM, D), lambda i: (i, 0)),
        out_shape=jax.ShapeDtypeStruct((B, D), jnp.float32),
    )(x, W_ext, b_ext2)


_RC = 128


def _km_body(f_ref, lab_ref, rank_ref):
    f = f_ref[...]
    fsq = jnp.sum(f * f, axis=1, keepdims=True)
    col_k = lax.broadcasted_iota(jnp.int32, (1, KP), 1)
    pad_mask = jnp.where(col_k >= K, jnp.float32(1e30), jnp.float32(0.0))
    oh_iota = lax.broadcasted_iota(jnp.int32, (B, KP), 1)

    def km_iter(_, carry):
        c, _ = carry
        d2 = (
            fsq
            - 2.0 * lax.dot_general(f, c, (((1,), (1,)), ((), ())),
                                    preferred_element_type=jnp.float32)
            + jnp.sum(c * c, axis=1)[None, :]
            + pad_mask
        )
        labels = jnp.argmin(d2, axis=1).astype(jnp.int32)
        oh = (labels[:, None] == oh_iota).astype(jnp.float32)
        sums = lax.dot_general(oh, f, (((0,), (0,)), ((), ())),
                               preferred_element_type=jnp.float32)
        counts = jnp.maximum(jnp.sum(oh, axis=0)[:, None], 1.0)
        return sums / counts, labels

    c0 = f[0:KP]
    _, labels = lax.fori_loop(
        0, KM_ITERS, km_iter, (c0, jnp.zeros((B,), jnp.int32))
    )

    oh = (labels[:, None] == oh_iota).astype(jnp.float32)
    lab_ref[...] = labels[:, None]

    k_iota = lax.broadcasted_iota(jnp.int32, (1, KP), 1).astype(jnp.float32)
    labels_row = lax.dot_general(k_iota, oh, (((1,), (1,)), ((), ())),
                                 preferred_element_type=jnp.float32)
    keys_row = (labels_row * B
                + lax.broadcasted_iota(jnp.int32, (1, B), 1).astype(jnp.float32))
    keys_col = (labels.astype(jnp.float32)[:, None] * B
                + lax.broadcasted_iota(jnp.int32, (B, 1), 0).astype(jnp.float32))
    rank = jnp.zeros((B, 1), jnp.float32)
    CH = 512
    for c0_ in range(0, B, CH):
        chunk = lax.slice(keys_row, (0, c0_), (1, c0_ + CH))
        rank = rank + jnp.sum((chunk < keys_col).astype(jnp.float32),
                              axis=1, keepdims=True)
    rank_ref[...] = rank.astype(jnp.int32)


def _tc_kmeans(feat):
    return pl.pallas_call(
        _km_body,
        out_shape=[
            jax.ShapeDtypeStruct((B, 1), jnp.int32),
            jax.ShapeDtypeStruct((B, 1), jnp.int32),
        ],
    )(feat)


def _est_body(f_ref, lab_ref, ws_ref, wsb_ref, bs_ref, pe_ref, z_ref):
    pe_proj = jnp.dot(pe_ref[...], ws_ref[...],
                      preferred_element_type=jnp.float32)
    lab_blk = lab_ref[...]
    oh_blk = (lab_blk
              == lax.broadcasted_iota(jnp.int32, (1, K), 1)).astype(jnp.float32)
    pe_add = lax.dot_general(oh_blk, pe_proj, (((1,), (0,)), ((), ())),
                             preferred_element_type=jnp.float32)
    z_ref[...] = (
        jnp.dot(f_ref[...].astype(jnp.bfloat16), wsb_ref[...],
                preferred_element_type=jnp.float32)
        + bs_ref[...]
        + pe_add
    )


def _tc_est(feat, labels_col, W_est, b_est2, pe_table):
    return pl.pallas_call(
        _est_body,
        grid=(B // _FM,),
        in_specs=[
            pl.BlockSpec((_FM, D), lambda i: (i, 0)),
            pl.BlockSpec((_FM, 1), lambda i: (i, 0)),
            pl.BlockSpec((D, D), lambda i: (0, 0)),
            pl.BlockSpec((D, D), lambda i: (0, 0)),
            pl.BlockSpec((1, D), lambda i: (0, 0)),
            pl.BlockSpec((K, D), lambda i: (0, 0)),
        ],
        out_specs=pl.BlockSpec((_FM, D), lambda i: (i, 0)),
        out_shape=jax.ShapeDtypeStruct((B, D), jnp.float32),
    )(feat, labels_col, W_est, W_est.astype(jnp.bfloat16), b_est2, pe_table)


_NW = 32
_PER = B // _NW
_CHUNK = 64


def _sc_permute(z, rank):
    mesh = plsc.VectorSubcoreMesh(core_axis_name="c", subcore_axis_name="s")

    @functools.partial(
        pl.kernel,
        out_type=jax.ShapeDtypeStruct((B, D), jnp.float32),
        mesh=mesh,
        scratch_types=[
            pltpu.VMEM((_CHUNK,), jnp.int32),
            pltpu.VMEM((_CHUNK, D), jnp.float32),
            pltpu.SemaphoreType.DMA,
        ],
    )
    def k(z_hbm, rank_hbm, out_hbm, idx_v, rows_v, sem):
        wid = lax.axis_index("s") * 2 + lax.axis_index("c")
        base = wid * _PER
        for c in range(_PER // _CHUNK):
            off = base + c * _CHUNK
            pltpu.sync_copy(rank_hbm.at[pl.ds(off, _CHUNK)], idx_v)
            pltpu.sync_copy(z_hbm.at[pl.ds(off, _CHUNK)], rows_v)
            pltpu.async_copy(rows_v, out_hbm.at[idx_v], sem).wait()

    return k(z, rank)


def kernel(x, W_ext, b_ext, W_est, b_est, pe_table):
    feat = _tc_feat(x, W_ext, b_ext.reshape(1, D))
    labels_col, rank_col = _tc_kmeans(feat)
    z = _tc_est(feat, labels_col, W_est, b_est.reshape(1, D), pe_table)
    return _sc_permute(z, rank_col.reshape(B))


# speedup vs baseline: 1.0193x; 1.0193x over previous
import functools

import jax
import jax.numpy as jnp
from jax import lax
from jax.experimental import pallas as pl
from jax.experimental.pallas import tpu as pltpu
from jax.experimental.pallas import tpu_sc as plsc

B = 4096
D = 1024
K = 4
KP = 8
KM_ITERS = 10


def _feat_body(x_ref, we_ref, be_ref, f_ref):
    f_ref[...] = jnp.maximum(
        jnp.dot(x_ref[...], we_ref[...], preferred_element_type=jnp.float32)
        + be_ref[...],
        0.0,
    )


_FM = 512


def _tc_feat(x, W_ext, b_ext2):
    return pl.pallas_call(
        _feat_body,
        grid=(B // _FM,),
        in_specs=[
            pl.BlockSpec((_FM, D), lambda i: (i, 0)),
            pl.BlockSpec((D, D), lambda i: (0, 0)),
            pl.BlockSpec((1, D), lambda i: (0, 0)),
        ],
        out_specs=pl.BlockSpec((_FM, D), lambda i: (i, 0)),
        out_shape=jax.ShapeDtypeStruct((B, D), jnp.float32),
    )(x, W_ext, b_ext2)


_RC = 128


def _km_body(f_ref, lab_ref, rank_ref):
    f = f_ref[...]
    fsq = jnp.sum(f * f, axis=1, keepdims=True)
    col_k = lax.broadcasted_iota(jnp.int32, (1, KP), 1)
    pad_mask = jnp.where(col_k >= K, jnp.float32(1e30), jnp.float32(0.0))
    oh_iota = lax.broadcasted_iota(jnp.int32, (B, KP), 1)

    def km_iter(_, carry):
        c, _ = carry
        d2 = (
            fsq
            - 2.0 * lax.dot_general(f, c, (((1,), (1,)), ((), ())),
                                    preferred_element_type=jnp.float32)
            + jnp.sum(c * c, axis=1)[None, :]
            + pad_mask
        )
        labels = jnp.argmin(d2, axis=1).astype(jnp.int32)
        oh = (labels[:, None] == oh_iota).astype(jnp.float32)
        sums = lax.dot_general(oh, f, (((0,), (0,)), ((), ())),
                               preferred_element_type=jnp.float32)
        counts = jnp.maximum(jnp.sum(oh, axis=0)[:, None], 1.0)
        return sums / counts, labels

    c0 = f[0:KP]
    _, labels = lax.fori_loop(
        0, KM_ITERS, km_iter, (c0, jnp.zeros((B,), jnp.int32))
    )

    oh = (labels[:, None] == oh_iota).astype(jnp.float32)
    k_iota = lax.broadcasted_iota(jnp.int32, (1, KP), 1).astype(jnp.float32)
    labels_row = lax.dot_general(k_iota, oh, (((1,), (1,)), ((), ())),
                                 preferred_element_type=jnp.float32)
    lab_ref[...] = labels_row.astype(jnp.int32)

    keys_row = (labels_row * B
                + lax.broadcasted_iota(jnp.int32, (1, B), 1).astype(jnp.float32))
    keys_col = (labels.astype(jnp.float32)[:, None] * B
                + lax.broadcasted_iota(jnp.int32, (B, 1), 0).astype(jnp.float32))
    rank = jnp.zeros((1, B), jnp.float32)
    CH = 256
    for r0 in range(0, B, CH):
        chunk = lax.slice(keys_col, (r0, 0), (r0 + CH, 1))
        rank = rank + jnp.sum((chunk < keys_row).astype(jnp.float32),
                              axis=0, keepdims=True)
    rank_ref[...] = rank.astype(jnp.int32)


def _tc_kmeans(feat):
    return pl.pallas_call(
        _km_body,
        out_shape=[
            jax.ShapeDtypeStruct((1, B), jnp.int32),
            jax.ShapeDtypeStruct((1, B), jnp.int32),
        ],
    )(feat)


def _est_body(f_ref, lab_ref, ws_ref, wsb_ref, bs_ref, pe_ref, z_ref):
    pe_proj = jnp.dot(pe_ref[...], ws_ref[...],
                      preferred_element_type=jnp.float32)
    lab_blk = lab_ref[...]
    oh_t = (lax.broadcasted_iota(jnp.int32, (K, 1), 0)
            == lab_blk).astype(jnp.float32)
    pe_add = lax.dot_general(oh_t, pe_proj, (((0,), (0,)), ((), ())),
                             preferred_element_type=jnp.float32)
    z_ref[...] = (
        jnp.dot(f_ref[...].astype(jnp.bfloat16), wsb_ref[...],
                preferred_element_type=jnp.float32)
        + bs_ref[...]
        + pe_add
    )


def _tc_est(feat, labels_col, W_est, b_est2, pe_table):
    return pl.pallas_call(
        _est_body,
        grid=(B // _FM,),
        in_specs=[
            pl.BlockSpec((_FM, D), lambda i: (i, 0)),
            pl.BlockSpec((1, _FM), lambda i: (0, i)),
            pl.BlockSpec((D, D), lambda i: (0, 0)),
            pl.BlockSpec((D, D), lambda i: (0, 0)),
            pl.BlockSpec((1, D), lambda i: (0, 0)),
            pl.BlockSpec((K, D), lambda i: (0, 0)),
        ],
        out_specs=pl.BlockSpec((_FM, D), lambda i: (i, 0)),
        out_shape=jax.ShapeDtypeStruct((B, D), jnp.float32),
    )(feat, labels_col, W_est, W_est.astype(jnp.bfloat16), b_est2, pe_table)


_NW = 32
_PER = B // _NW
_CHUNK = 64


def _sc_permute(z, rank):
    mesh = plsc.VectorSubcoreMesh(core_axis_name="c", subcore_axis_name="s")

    @functools.partial(
        pl.kernel,
        out_type=jax.ShapeDtypeStruct((B, D), jnp.float32),
        mesh=mesh,
        scratch_types=[
            pltpu.VMEM((_CHUNK,), jnp.int32),
            pltpu.VMEM((_CHUNK, D), jnp.float32),
            pltpu.SemaphoreType.DMA,
        ],
    )
    def k(z_hbm, rank_hbm, out_hbm, idx_v, rows_v, sem):
        wid = lax.axis_index("s") * 2 + lax.axis_index("c")
        base = wid * _PER
        for c in range(_PER // _CHUNK):
            off = base + c * _CHUNK
            pltpu.sync_copy(rank_hbm.at[pl.ds(off, _CHUNK)], idx_v)
            pltpu.sync_copy(z_hbm.at[pl.ds(off, _CHUNK)], rows_v)
            pltpu.async_copy(rows_v, out_hbm.at[idx_v], sem).wait()

    return k(z, rank)


def kernel(x, W_ext, b_ext, W_est, b_est, pe_table):
    feat = _tc_feat(x, W_ext, b_ext.reshape(1, D))
    labels_col, rank_col = _tc_kmeans(feat)
    z = _tc_est(feat, labels_col, W_est, b_est.reshape(1, D), pe_table)
    return _sc_permute(z, rank_col.reshape(B))


# speedup vs baseline: 1.0264x; 1.0070x over previous
import functools

import jax
import jax.numpy as jnp
from jax import lax
from jax.experimental import pallas as pl
from jax.experimental.pallas import tpu as pltpu
from jax.experimental.pallas import tpu_sc as plsc

B = 4096
D = 1024
K = 4
KP = 8
KM_ITERS = 10


def _feat_body(x_ref, we_ref, be_ref, f_ref):
    f_ref[...] = jnp.maximum(
        jnp.dot(x_ref[...], we_ref[...], preferred_element_type=jnp.float32)
        + be_ref[...],
        0.0,
    )


_FM = 512


def _tc_feat(x, W_ext, b_ext2):
    return pl.pallas_call(
        _feat_body,
        grid=(B // _FM,),
        in_specs=[
            pl.BlockSpec((_FM, D), lambda i: (i, 0)),
            pl.BlockSpec((D, D), lambda i: (0, 0)),
            pl.BlockSpec((1, D), lambda i: (0, 0)),
        ],
        out_specs=pl.BlockSpec((_FM, D), lambda i: (i, 0)),
        out_shape=jax.ShapeDtypeStruct((B, D), jnp.float32),
    )(x, W_ext, b_ext2)


_RC = 128


def _km_body(f_ref, ws_ref, bs_ref, pe_ref, lab_ref, rank_ref, pep_ref):
    pep_ref[...] = (
        jnp.dot(pe_ref[...], ws_ref[...], preferred_element_type=jnp.float32)
        + bs_ref[...]
    )
    f = f_ref[...]
    fsq = jnp.sum(f * f, axis=1, keepdims=True)
    col_k = lax.broadcasted_iota(jnp.int32, (1, KP), 1)
    pad_mask = jnp.where(col_k >= K, jnp.float32(1e30), jnp.float32(0.0))
    oh_iota = lax.broadcasted_iota(jnp.int32, (B, KP), 1)

    def km_iter(_, carry):
        c, _ = carry
        d2 = (
            fsq
            - 2.0 * lax.dot_general(f, c, (((1,), (1,)), ((), ())),
                                    preferred_element_type=jnp.float32)
            + jnp.sum(c * c, axis=1)[None, :]
            + pad_mask
        )
        labels = jnp.argmin(d2, axis=1).astype(jnp.int32)
        oh = (labels[:, None] == oh_iota).astype(jnp.float32)
        sums = lax.dot_general(oh, f, (((0,), (0,)), ((), ())),
                               preferred_element_type=jnp.float32)
        counts = jnp.maximum(jnp.sum(oh, axis=0)[:, None], 1.0)
        return sums / counts, labels

    c0 = f[0:KP]
    _, labels = lax.fori_loop(
        0, KM_ITERS, km_iter, (c0, jnp.zeros((B,), jnp.int32))
    )

    oh = (labels[:, None] == oh_iota).astype(jnp.float32)
    k_iota = lax.broadcasted_iota(jnp.int32, (1, KP), 1).astype(jnp.float32)
    labels_row = lax.dot_general(k_iota, oh, (((1,), (1,)), ((), ())),
                                 preferred_element_type=jnp.float32)
    lab_ref[...] = labels_row.astype(jnp.int32)

    keys_row = (labels_row * B
                + lax.broadcasted_iota(jnp.int32, (1, B), 1).astype(jnp.float32))
    keys_col = (labels.astype(jnp.float32)[:, None] * B
                + lax.broadcasted_iota(jnp.int32, (B, 1), 0).astype(jnp.float32))
    rank = jnp.zeros((1, B), jnp.float32)
    CH = 256
    for r0 in range(0, B, CH):
        chunk = lax.slice(keys_col, (r0, 0), (r0 + CH, 1))
        rank = rank + jnp.sum((chunk < keys_row).astype(jnp.float32),
                              axis=0, keepdims=True)
    rank_ref[...] = rank.astype(jnp.int32)


def _tc_kmeans(feat, W_est, b_est2, pe_table):
    return pl.pallas_call(
        _km_body,
        out_shape=[
            jax.ShapeDtypeStruct((1, B), jnp.int32),
            jax.ShapeDtypeStruct((1, B), jnp.int32),
            jax.ShapeDtypeStruct((K, D), jnp.float32),
        ],
    )(feat, W_est, b_est2, pe_table)


def _est_body(f_ref, lab_ref, ws_ref, pep_ref, z_ref):
    lab_blk = lab_ref[...]
    oh_t = (lax.broadcasted_iota(jnp.int32, (K, 1), 0)
            == lab_blk).astype(jnp.float32)
    pe_add = lax.dot_general(oh_t, pep_ref[...], (((0,), (0,)), ((), ())),
                             preferred_element_type=jnp.float32)
    z_ref[...] = (
        jnp.dot(f_ref[...], ws_ref[...], preferred_element_type=jnp.float32)
        + pe_add
    )


def _tc_est(feat, labels_row, W_est, pe_projb):
    return pl.pallas_call(
        _est_body,
        grid=(B // _FM,),
        in_specs=[
            pl.BlockSpec((_FM, D), lambda i: (i, 0)),
            pl.BlockSpec((1, _FM), lambda i: (0, i)),
            pl.BlockSpec((D, D), lambda i: (0, 0)),
            pl.BlockSpec((K, D), lambda i: (0, 0)),
        ],
        out_specs=pl.BlockSpec((_FM, D), lambda i: (i, 0)),
        out_shape=jax.ShapeDtypeStruct((B, D), jnp.float32),
    )(feat, labels_row, W_est, pe_projb)


_NW = 32
_PER = B // _NW
_CHUNK = 32
_NCH = _PER // _CHUNK


def _sc_permute(z, rank2d):
    mesh = plsc.VectorSubcoreMesh(core_axis_name="c", subcore_axis_name="s")

    @functools.partial(
        pl.kernel,
        out_type=jax.ShapeDtypeStruct((B, D), jnp.float32),
        mesh=mesh,
        scratch_types=[
            pltpu.VMEM((_NCH, _CHUNK), jnp.int32),
            pltpu.VMEM((_CHUNK, D), jnp.float32),
            pltpu.VMEM((_CHUNK, D), jnp.float32),
            pltpu.SemaphoreType.DMA,
            pltpu.SemaphoreType.DMA,
            pltpu.SemaphoreType.DMA,
        ],
    )
    def k(z_hbm, rank_hbm, out_hbm, idx_v, row0, row1, ls0, ls1, ssem):
        wid = lax.axis_index("s") * 2 + lax.axis_index("c")
        base = wid * _PER
        bufs = (row0, row1)
        lsems = (ls0, ls1)
        pltpu.sync_copy(rank_hbm.at[pl.ds(wid * _NCH, _NCH)], idx_v)
        loads = [pltpu.async_copy(z_hbm.at[pl.ds(base, _CHUNK)], row0, ls0)]
        for c in range(_NCH):
            if c + 1 < _NCH:
                loads.append(pltpu.async_copy(
                    z_hbm.at[pl.ds(base + (c + 1) * _CHUNK, _CHUNK)],
                    bufs[(c + 1) % 2], lsems[(c + 1) % 2]))
            loads[c].wait()
            pltpu.async_copy(bufs[c % 2], out_hbm.at[idx_v.at[c]], ssem).wait()

    return k(z, rank2d)


def kernel(x, W_ext, b_ext, W_est, b_est, pe_table):
    feat = _tc_feat(x, W_ext, b_ext.reshape(1, D))
    labels_row, rank_row, pe_projb = _tc_kmeans(
        feat, W_est, b_est.reshape(1, D), pe_table)
    z = _tc_est(feat, labels_row, W_est, pe_projb)
    return _sc_permute(z, rank_row.reshape(B // _CHUNK, _CHUNK))


# speedup vs baseline: 1.0486x; 1.0216x over previous
import functools

import jax
import jax.numpy as jnp
from jax import lax
from jax.experimental import pallas as pl
from jax.experimental.pallas import tpu as pltpu
from jax.experimental.pallas import tpu_sc as plsc

B = 4096
D = 1024
K = 4
KP = 8
KM_ITERS = 10


def _feat_body(x_ref, we_ref, be_ref, f_ref):
    f_ref[...] = jnp.maximum(
        jnp.dot(x_ref[...], we_ref[...], preferred_element_type=jnp.float32)
        + be_ref[...],
        0.0,
    )


_FM = 512


def _tc_feat(x, W_ext, b_ext2):
    return pl.pallas_call(
        _feat_body,
        grid=(B // _FM,),
        in_specs=[
            pl.BlockSpec((_FM, D), lambda i: (i, 0)),
            pl.BlockSpec((D, D), lambda i: (0, 0)),
            pl.BlockSpec((1, D), lambda i: (0, 0)),
        ],
        out_specs=pl.BlockSpec((_FM, D), lambda i: (i, 0)),
        out_shape=jax.ShapeDtypeStruct((B, D), jnp.float32),
    )(x, W_ext, b_ext2)


_RC = 128


def _km_body(f_ref, ws_ref, bs_ref, pe_ref, lab_ref, rank_ref, pep_ref):
    pep_ref[...] = (
        jnp.dot(pe_ref[...], ws_ref[...], preferred_element_type=jnp.float32)
        + bs_ref[...]
    )
    f = f_ref[...]
    fsq = jnp.sum(f * f, axis=1, keepdims=True)
    col_k = lax.broadcasted_iota(jnp.int32, (1, KP), 1)
    pad_mask = jnp.where(col_k >= K, jnp.float32(1e30), jnp.float32(0.0))
    oh_iota = lax.broadcasted_iota(jnp.int32, (B, KP), 1)

    def km_iter(_, carry):
        c, _ = carry
        d2 = (
            fsq
            - 2.0 * lax.dot_general(f, c, (((1,), (1,)), ((), ())),
                                    preferred_element_type=jnp.float32)
            + jnp.sum(c * c, axis=1)[None, :]
            + pad_mask
        )
        labels = jnp.argmin(d2, axis=1).astype(jnp.int32)
        oh = (labels[:, None] == oh_iota).astype(jnp.float32)
        sums = lax.dot_general(oh, f, (((0,), (0,)), ((), ())),
                               preferred_element_type=jnp.float32)
        counts = jnp.maximum(jnp.sum(oh, axis=0)[:, None], 1.0)
        return sums / counts, labels

    c0 = f[0:KP]
    _, labels = lax.fori_loop(
        0, KM_ITERS, km_iter, (c0, jnp.zeros((B,), jnp.int32))
    )

    oh = (labels[:, None] == oh_iota).astype(jnp.float32)
    k_iota = lax.broadcasted_iota(jnp.int32, (1, KP), 1).astype(jnp.float32)
    labels_row = lax.dot_general(k_iota, oh, (((1,), (1,)), ((), ())),
                                 preferred_element_type=jnp.float32)
    lab_ref[...] = labels_row.astype(jnp.int32)

    keys_row = (labels_row * B
                + lax.broadcasted_iota(jnp.int32, (1, B), 1).astype(jnp.float32))
    keys_col = (labels.astype(jnp.float32)[:, None] * B
                + lax.broadcasted_iota(jnp.int32, (B, 1), 0).astype(jnp.float32))
    rank = jnp.zeros((1, B), jnp.float32)
    CH = 256
    for r0 in range(0, B, CH):
        chunk = lax.slice(keys_col, (r0, 0), (r0 + CH, 1))
        rank = rank + jnp.sum((chunk < keys_row).astype(jnp.float32),
                              axis=0, keepdims=True)
    rank_ref[...] = rank.astype(jnp.int32)


def _tc_kmeans(feat, W_est, b_est2, pe_table):
    return pl.pallas_call(
        _km_body,
        out_shape=[
            jax.ShapeDtypeStruct((1, B), jnp.int32),
            jax.ShapeDtypeStruct((1, B), jnp.int32),
            jax.ShapeDtypeStruct((K, D), jnp.float32),
        ],
    )(feat, W_est, b_est2, pe_table)


def _est_body(f_ref, lab_ref, ws_ref, pep_ref, z_ref):
    lab_blk = lab_ref[...]
    oh_t = (lax.broadcasted_iota(jnp.int32, (K, 1), 0)
            == lab_blk).astype(jnp.float32)
    pe_add = lax.dot_general(oh_t, pep_ref[...], (((0,), (0,)), ((), ())),
                             preferred_element_type=jnp.float32)
    z_ref[...] = (
        jnp.dot(f_ref[...], ws_ref[...], preferred_element_type=jnp.float32)
        + pe_add
    )


def _tc_est(feat, labels_row, W_est, pe_projb):
    return pl.pallas_call(
        _est_body,
        grid=(B // _FM,),
        in_specs=[
            pl.BlockSpec((_FM, D), lambda i: (i, 0)),
            pl.BlockSpec((1, _FM), lambda i: (0, i)),
            pl.BlockSpec((D, D), lambda i: (0, 0)),
            pl.BlockSpec((K, D), lambda i: (0, 0)),
        ],
        out_specs=pl.BlockSpec((_FM, D), lambda i: (i, 0)),
        out_shape=jax.ShapeDtypeStruct((B, D), jnp.float32),
    )(feat, labels_row, W_est, pe_projb)


_NW = 32
_PER = B // _NW
_CHUNK = 64


def _sc_permute(z, rank):
    mesh = plsc.VectorSubcoreMesh(core_axis_name="c", subcore_axis_name="s")

    @functools.partial(
        pl.kernel,
        out_type=jax.ShapeDtypeStruct((B, D), jnp.float32),
        mesh=mesh,
        scratch_types=[
            pltpu.VMEM((_CHUNK,), jnp.int32),
            pltpu.VMEM((_CHUNK, D), jnp.float32),
            pltpu.SemaphoreType.DMA,
        ],
    )
    def k(z_hbm, rank_hbm, out_hbm, idx_v, rows_v, sem):
        wid = lax.axis_index("s") * 2 + lax.axis_index("c")
        base = wid * _PER
        for c in range(_PER // _CHUNK):
            off = base + c * _CHUNK
            pltpu.sync_copy(rank_hbm.at[pl.ds(off, _CHUNK)], idx_v)
            pltpu.sync_copy(z_hbm.at[pl.ds(off, _CHUNK)], rows_v)
            pltpu.async_copy(rows_v, out_hbm.at[idx_v], sem).wait()

    return k(z, rank)


def kernel(x, W_ext, b_ext, W_est, b_est, pe_table):
    feat = _tc_feat(x, W_ext, b_ext.reshape(1, D))
    labels_row, rank_row, pe_projb = _tc_kmeans(
        feat, W_est, b_est.reshape(1, D), pe_table)
    z = _tc_est(feat, labels_row, W_est, pe_projb)
    return _sc_permute(z, rank_row.reshape(B))
